# baseline (speedup 1.0000x reference)
import jax
import jax.numpy as jnp
from jax import lax
from jax.experimental import pallas as pl
from jax.experimental.pallas import tpu as pltpu
from jax.experimental.pallas import tpu_sc as plsc

_NC = 2
_NS = 16
_L = 16
_LN2 = 0.6931471805599453
_SQRT2 = 1.4142135381698608


def _logsig_eps(s):
    z = jnp.exp(-s)
    y = 1.0 / (1.0 + z) + 1e-9
    bits = lax.bitcast_convert_type(y, jnp.int32)
    e = (bits >> 23) - 127
    m = lax.bitcast_convert_type((bits & 0x007FFFFF) | 0x3F800000, jnp.float32)
    big = m > _SQRT2
    m = jnp.where(big, m * 0.5, m)
    e = jnp.where(big, e + 1, e)
    t = (m - 1.0) / (m + 1.0)
    t2 = t * t
    p = 1.0 + t2 * (0.3333333333 + t2 * (0.2 + t2 * (0.142857143 + t2 * 0.111111111)))
    return e.astype(jnp.float32) * _LN2 + 2.0 * t * p


def _make_sgns_kernel(B, K, V, D):
    NW = _NC * _NS
    assert B % NW == 0
    BPW = B // NW
    CH = 16 if BPW % 16 == 0 else BPW
    assert BPW % CH == 0 and D % _L == 0
    NCHUNK = BPW // CH
    KP1 = K + 1
    NDOT = CH * KP1
    assert NDOT % _L == 0
    ND = D // _L
    assert (CH * K) % _L == 0
    mesh = plsc.VectorSubcoreMesh(
        core_axis_name="cores", subcore_axis_name="subcores",
        num_cores=_NC, num_subcores=_NS)

    assert NCHUNK % 2 == 0
    NSTEP = NCHUNK // 2

    def body(c_hbm, o_hbm, neg_hbm, v_hbm, u_hbm, out_hbm,
             c_idx, o_idx, n_idx, vc_b0, uo_b0, un_b0, vc_b1, uo_b1, un_b1,
             trans, acc, sem0, sem1):
        wid = lax.axis_index("subcores") * _NC + lax.axis_index("cores")
        b0 = wid * BPW
        pltpu.sync_copy(c_hbm.at[pl.ds(b0, BPW)], c_idx)
        pltpu.sync_copy(o_hbm.at[pl.ds(b0, BPW)], o_idx)
        pltpu.sync_copy(neg_hbm.at[pl.ds(b0 * K, BPW * K)], n_idx)
        acc[...] = jnp.zeros((_L,), jnp.float32)

        def issue(i, vcb, uob, unb, sem):
            cb = i * CH
            nb = i * (CH * K)
            pltpu.async_copy(v_hbm.at[c_idx[pl.ds(cb, CH)]], vcb, sem)
            pltpu.async_copy(u_hbm.at[o_idx[pl.ds(cb, CH)]], uob, sem)
            for g in range(CH * K // _L):
                pltpu.async_copy(
                    u_hbm.at[n_idx[pl.ds(nb + g * _L, _L)]],
                    unb.at[pl.ds(g * _L, _L)], sem)

        def drain(vcb, uob, unb, sem):
            pltpu.make_async_copy(v_hbm.at[pl.ds(0, CH)], vcb, sem).wait()
            pltpu.make_async_copy(u_hbm.at[pl.ds(0, CH)], uob, sem).wait()
            pltpu.make_async_copy(u_hbm.at[pl.ds(0, CH * K)], unb, sem).wait()

        lane = lax.iota(jnp.int32, _L)
        CPB = 12
        W32 = CH * CPB
        lo8 = lane < 8
        rowbase = (lane & 7) * W32

        def col_tables(G):
            m = lax.rem(lane + _L * G, CPB)
            lo_valid = m <= 10
            hi_valid = m <= 9
            sign_lo = jnp.where(m == 0, 1.0, -1.0)
            return sign_lo, lo_valid, hi_valid

        def fold2(a_, b_):
            return jnp.where(lo8, a_ + lax.rev(a_, (0,)), b_ + lax.rev(b_, (0,)))

        def compute(vcb, uob, unb):
            def per_b(b, _):
                base = rowbase + b * CPB
                vc = [vcb[b, pl.ds(_L * j, _L)] for j in range(ND)]
                pp = vc[0] * uob[b, pl.ds(0, _L)]
                for j in range(1, ND):
                    pp = pp + vc[j] * uob[b, pl.ds(_L * j, _L)]
                nns = []
                for k in range(K):
                    r = b * K + k
                    nn = vc[0] * unb[r, pl.ds(0, _L)]
                    for j in range(1, ND):
                        nn = nn + vc[j] * unb[r, pl.ds(_L * j, _L)]
                    nns.append(nn)
                ds_ = [pp] + nns
                for t in range(5):
                    fa = fold2(ds_[4 * t], ds_[4 * t + 1])
                    fb = fold2(ds_[4 * t + 2], ds_[4 * t + 3])
                    w = plsc.bitcast(
                        plsc.pack(fa, fb, format=plsc.PackFormat.INTERLEAVED),
                        jnp.float32)
                    off = jnp.where(lo8, 2 * t, 2 * t + 1)
                    plsc.store_scatter(trans, [base + off], w)
                last = ds_[K]
                fl = last + lax.rev(last, (0,))
                w = plsc.bitcast(
                    plsc.pack(fl, jnp.zeros((_L,), jnp.float32),
                              format=plsc.PackFormat.INTERLEAVED),
                    jnp.float32)
                off = jnp.where(lo8, 10, 11)
                plsc.store_scatter(trans, [base + off], w)
                return 0

            lax.fori_loop(0, CH, per_b, 0)
            a = acc[...]
            for G in range(W32 // _L):
                sign_lo, lo_valid, hi_valid = col_tables(G)
                los, his = [], []
                for j in range(8):
                    w = trans[pl.ds(j * W32 + _L * G, _L)]
                    pa, pb = plsc.unpack(
                        plsc.bitcast(w, jnp.bfloat16),
                        format=plsc.PackFormat.INTERLEAVED)
                    los.append(pa)
                    his.append(pb)
                while len(los) > 1:
                    los = [los[2 * i] + los[2 * i + 1]
                           for i in range(len(los) // 2)]
                    his = [his[2 * i] + his[2 * i + 1]
                           for i in range(len(his) // 2)]
                zero = jnp.zeros((_L,), jnp.float32)
                a = a + jnp.where(lo_valid,
                                  _logsig_eps(los[0] * sign_lo), zero)
                a = a + jnp.where(hi_valid, _logsig_eps(-his[0]), zero)
            acc[...] = a

        issue(0, vc_b0, uo_b0, un_b0, sem0)

        def step(s, _):
            issue(2 * s + 1, vc_b1, uo_b1, un_b1, sem1)
            drain(vc_b0, uo_b0, un_b0, sem0)
            compute(vc_b0, uo_b0, un_b0)

            @pl.when(s + 1 < NSTEP)
            def _():
                issue(2 * s + 2, vc_b0, uo_b0, un_b0, sem0)

            drain(vc_b1, uo_b1, un_b1, sem1)
            compute(vc_b1, uo_b1, un_b1)
            return 0

        lax.fori_loop(0, NSTEP, step, 0)
        pltpu.sync_copy(acc, out_hbm.at[wid])

    return pl.kernel(
        body,
        out_type=jax.ShapeDtypeStruct((NW, _L), jnp.float32),
        mesh=mesh,
        compiler_params=pltpu.CompilerParams(needs_layout_passes=False),
        scratch_types=[
            pltpu.VMEM((BPW,), jnp.int32),
            pltpu.VMEM((BPW,), jnp.int32),
            pltpu.VMEM((BPW * K,), jnp.int32),
            pltpu.VMEM((CH, D), jnp.float32),
            pltpu.VMEM((CH, D), jnp.float32),
            pltpu.VMEM((CH * K, D), jnp.float32),
            pltpu.VMEM((CH, D), jnp.float32),
            pltpu.VMEM((CH, D), jnp.float32),
            pltpu.VMEM((CH * K, D), jnp.float32),
            pltpu.VMEM((8 * CH * 12,), jnp.float32),
            pltpu.VMEM((_L,), jnp.float32),
            pltpu.SemaphoreType.DMA,
            pltpu.SemaphoreType.DMA,
        ],
    )


def kernel(c, o, neg, V_emb, U_emb):
    (B,) = c.shape
    K = neg.shape[1]
    V, D = V_emb.shape
    c = c.astype(jnp.int32)
    o = o.astype(jnp.int32)
    negf = neg.reshape(-1).astype(jnp.int32)
    fn = _make_sgns_kernel(B, K, V, D)
    partials = fn(c, o, negf, V_emb, U_emb)
    return -(jnp.sum(partials) / B)


# speedup vs baseline: 1.0356x; 1.0356x over previous
import jax
import jax.numpy as jnp
from jax import lax
from jax.experimental import pallas as pl
from jax.experimental.pallas import tpu as pltpu
from jax.experimental.pallas import tpu_sc as plsc

_NC = 2
_NS = 16
_L = 16
_LN2 = 0.6931471805599453
_SQRT2 = 1.4142135381698608


def _logsig_eps(s):
    z = jnp.exp(-s)
    y = 1.0 / (1.0 + z) + 1e-9
    bits = lax.bitcast_convert_type(y, jnp.int32)
    e = (bits >> 23) - 127
    m = lax.bitcast_convert_type((bits & 0x007FFFFF) | 0x3F800000, jnp.float32)
    big = m > _SQRT2
    m = jnp.where(big, m * 0.5, m)
    e = jnp.where(big, e + 1, e)
    t = (m - 1.0) / (m + 1.0)
    t2 = t * t
    p = 1.0 + t2 * (0.3333333333 + t2 * (0.2 + t2 * (0.142857143 + t2 * 0.111111111)))
    return e.astype(jnp.float32) * _LN2 + 2.0 * t * p


def _make_sgns_kernel(B, K, V, D):
    NW = _NC * _NS
    assert B % NW == 0
    BPW = B // NW
    CH = 16 if BPW % 16 == 0 else BPW
    assert BPW % CH == 0 and D % _L == 0
    NCHUNK = BPW // CH
    KP1 = K + 1
    NDOT = CH * KP1
    assert NDOT % _L == 0
    ND = D // _L
    assert (CH * K) % _L == 0
    mesh = plsc.VectorSubcoreMesh(
        core_axis_name="cores", subcore_axis_name="subcores",
        num_cores=_NC, num_subcores=_NS)

    assert NCHUNK % 2 == 0
    NSTEP = NCHUNK // 2

    def body(c_hbm, o_hbm, neg_hbm, v_hbm, u_hbm, out_hbm,
             c_idx, o_idx, n_idx, vc_b0, uo_b0, un_b0, vc_b1, uo_b1, un_b1,
             trans, acc, sem0, sem1):
        wid = lax.axis_index("subcores") * _NC + lax.axis_index("cores")
        b0 = wid * BPW
        pltpu.sync_copy(c_hbm.at[pl.ds(b0, BPW)], c_idx)
        pltpu.sync_copy(o_hbm.at[pl.ds(b0, BPW)], o_idx)
        pltpu.sync_copy(neg_hbm.at[pl.ds(b0 * K, BPW * K)], n_idx)
        acc[...] = jnp.zeros((_L,), jnp.float32)

        def issue(i, vcb, uob, unb, sem):
            cb = i * CH
            nb = i * (CH * K)
            pltpu.async_copy(v_hbm.at[c_idx[pl.ds(cb, CH)]], vcb, sem)
            pltpu.async_copy(u_hbm.at[o_idx[pl.ds(cb, CH)]], uob, sem)
            done = 0
            while done < CH * K:
                n = min(128, CH * K - done)
                pltpu.async_copy(
                    u_hbm.at[n_idx.at[pl.ds(nb + done, n)]],
                    unb.at[pl.ds(done, n)], sem)
                done += n

        def drain(vcb, uob, unb, sem):
            pltpu.make_async_copy(v_hbm.at[pl.ds(0, CH)], vcb, sem).wait()
            pltpu.make_async_copy(u_hbm.at[pl.ds(0, CH)], uob, sem).wait()
            pltpu.make_async_copy(u_hbm.at[pl.ds(0, CH * K)], unb, sem).wait()

        lane = lax.iota(jnp.int32, _L)
        NDOTC = KP1 * _L
        TW = NDOTC + 8
        DUMP = NDOTC
        lo8 = lane < 8
        rowbase = (lane & 7) * TW

        def fold2(a_, b_):
            return jnp.where(lo8, a_ + lax.rev(a_, (0,)), b_ + lax.rev(b_, (0,)))

        def compute(vcb, uob, unb):
            def per_b(b, _):
                d0 = b * KP1
                base = rowbase + d0
                vc = [vcb[b, pl.ds(_L * j, _L)] for j in range(ND)]
                pp = vc[0] * uob[b, pl.ds(0, _L)]
                for j in range(1, ND):
                    pp = pp + vc[j] * uob[b, pl.ds(_L * j, _L)]
                nns = []
                for k in range(K):
                    r = b * K + k
                    nn = vc[0] * unb[r, pl.ds(0, _L)]
                    for j in range(1, ND):
                        nn = nn + vc[j] * unb[r, pl.ds(_L * j, _L)]
                    nns.append(nn)
                ds_ = [pp] + nns
                for p_ in range(KP1 // 2):
                    a_, b_ = ds_[2 * p_], ds_[2 * p_ + 1]
                    off = jnp.where(lo8, 2 * p_, 2 * p_ + 1)
                    plsc.store_scatter(trans, [base + off], fold2(a_, b_))
                last = ds_[KP1 - 1]
                off = jnp.where(lo8, jnp.full((_L,), KP1 - 1, jnp.int32),
                                DUMP - d0 + lane * 0)
                plsc.store_scatter(
                    trans, [base + off],
                    jnp.where(lo8, last + lax.rev(last, (0,)), last))
                return 0

            lax.fori_loop(0, CH, per_b, 0)
            a = acc[...]
            for g in range(KP1):
                vs = [trans[pl.ds(j * TW + g * _L, _L)] for j in range(8)]
                while len(vs) > 1:
                    vs = [vs[2 * i] + vs[2 * i + 1]
                          for i in range(len(vs) // 2)]
                d_idx = lane + g * _L
                s = jnp.where(d_idx % KP1 == 0, vs[0], -vs[0])
                a = a + _logsig_eps(s)
            acc[...] = a

        issue(0, vc_b0, uo_b0, un_b0, sem0)

        def step(s, _):
            issue(2 * s + 1, vc_b1, uo_b1, un_b1, sem1)
            drain(vc_b0, uo_b0, un_b0, sem0)
            compute(vc_b0, uo_b0, un_b0)

            @pl.when(s + 1 < NSTEP)
            def _():
                issue(2 * s + 2, vc_b0, uo_b0, un_b0, sem0)

            drain(vc_b1, uo_b1, un_b1, sem1)
            compute(vc_b1, uo_b1, un_b1)
            return 0

        lax.fori_loop(0, NSTEP, step, 0)
        pltpu.sync_copy(acc, out_hbm.at[wid])

    return pl.kernel(
        body,
        out_type=jax.ShapeDtypeStruct((NW, _L), jnp.float32),
        mesh=mesh,
        compiler_params=pltpu.CompilerParams(needs_layout_passes=False),
        scratch_types=[
            pltpu.VMEM((BPW,), jnp.int32),
            pltpu.VMEM((BPW,), jnp.int32),
            pltpu.VMEM((BPW * K,), jnp.int32),
            pltpu.VMEM((CH, D), jnp.float32),
            pltpu.VMEM((CH, D), jnp.float32),
            pltpu.VMEM((CH * K, D), jnp.float32),
            pltpu.VMEM((CH, D), jnp.float32),
            pltpu.VMEM((CH, D), jnp.float32),
            pltpu.VMEM((CH * K, D), jnp.float32),
            pltpu.VMEM((8 * (NDOT + 8),), jnp.float32),
            pltpu.VMEM((_L,), jnp.float32),
            pltpu.SemaphoreType.DMA,
            pltpu.SemaphoreType.DMA,
        ],
    )


def kernel(c, o, neg, V_emb, U_emb):
    (B,) = c.shape
    K = neg.shape[1]
    V, D = V_emb.shape
    c = c.astype(jnp.int32)
    o = o.astype(jnp.int32)
    negf = neg.reshape(-1).astype(jnp.int32)
    fn = _make_sgns_kernel(B, K, V, D)
    partials = fn(c, o, negf, V_emb, U_emb)
    return -(jnp.sum(partials) / B)
